# baseline (device time: 1386755 ns/iter reference)
import jax
import jax.numpy as jnp
from jax import lax
from jax.experimental import pallas as pl
from jax.experimental.pallas import tpu as pltpu

NZ = 4


def kernel(ids, E):
    v_local, d = E.shape
    t = ids.shape[0]

    my_z = lax.axis_index("z")
    local = ids - my_z * v_local
    in_range = (local >= 0) & (local < v_local)
    partial = jnp.where(
        in_range[:, None], E[jnp.where(in_range, local, 0)], 0.0
    ).astype(jnp.float32)

    def body(x_ref, out_ref, comm_ref, send_sems, recv_sems):
        my_x = lax.axis_index("x")
        my_y = lax.axis_index("y")
        mz = lax.axis_index("z")
        left = lax.rem(mz + NZ - 1, NZ)
        right = lax.rem(mz + 1, NZ)

        barrier_sem = pltpu.get_barrier_semaphore()
        for nbr in (left, right):
            pl.semaphore_signal(
                barrier_sem, inc=1,
                device_id=(my_x, my_y, nbr),
                device_id_type=pl.DeviceIdType.MESH,
            )
        pl.semaphore_wait(barrier_sem, 2)

        out_ref[...] = x_ref[...]

        for h in range(NZ - 1):
            src = x_ref if h == 0 else comm_ref.at[h - 1]
            rdma = pltpu.make_async_remote_copy(
                src_ref=src,
                dst_ref=comm_ref.at[h],
                send_sem=send_sems.at[h],
                recv_sem=recv_sems.at[h],
                device_id=(my_x, my_y, right),
                device_id_type=pl.DeviceIdType.MESH,
            )
            rdma.start()
            rdma.wait()
            out_ref[...] += comm_ref[h]

    return pl.pallas_call(
        body,
        out_shape=jax.ShapeDtypeStruct((t, d), jnp.float32),
        in_specs=[pl.BlockSpec(memory_space=pltpu.VMEM)],
        out_specs=pl.BlockSpec(memory_space=pltpu.VMEM),
        scratch_shapes=[
            pltpu.VMEM((NZ - 1, t, d), jnp.float32),
            pltpu.SemaphoreType.DMA((NZ - 1,)),
            pltpu.SemaphoreType.DMA((NZ - 1,)),
        ],
        compiler_params=pltpu.CompilerParams(collective_id=0),
    )(partial)


# device time: 370934 ns/iter; 3.7385x vs baseline; 3.7385x over previous
import jax
import jax.numpy as jnp
from jax import lax
from jax.experimental import pallas as pl
from jax.experimental.pallas import tpu as pltpu

NZ = 4
K = 16


def _gather_partial(safe, msk, E, t, d):

    def body(idx_ref, msk_ref, e_ref, o_ref, sems):
        o_ref[...] = jnp.zeros_like(o_ref)

        def cp(tk):
            return pltpu.make_async_copy(
                e_ref.at[idx_ref[tk]], o_ref.at[tk], sems.at[lax.rem(tk, K)]
            )

        def loop(tk, c):
            @pl.when((tk >= K) & (msk_ref[tk - K] == 1))
            def _():
                cp(tk - K).wait()

            @pl.when(msk_ref[tk] == 1)
            def _():
                cp(tk).start()

            return c

        lax.fori_loop(0, t, loop, 0)

        def ep(i, c):
            tk = t - K + i

            @pl.when(msk_ref[tk] == 1)
            def _():
                cp(tk).wait()

            return c

        lax.fori_loop(0, K, ep, 0)

    return pl.pallas_call(
        body,
        out_shape=jax.ShapeDtypeStruct((t, d), jnp.float32),
        in_specs=[
            pl.BlockSpec(memory_space=pltpu.SMEM),
            pl.BlockSpec(memory_space=pltpu.SMEM),
            pl.BlockSpec(memory_space=pl.ANY),
        ],
        out_specs=pl.BlockSpec(memory_space=pltpu.VMEM),
        scratch_shapes=[pltpu.SemaphoreType.DMA((K,))],
    )(safe, msk, E)


def kernel(ids, E):
    v_local, d = E.shape
    t = ids.shape[0]

    my_z = lax.axis_index("z")
    local = ids - my_z * v_local
    in_range = (local >= 0) & (local < v_local)
    safe = jnp.where(in_range, local, 0).astype(jnp.int32)
    msk = in_range.astype(jnp.int32)
    partial = _gather_partial(safe, msk, E, t, d)

    def body(x_ref, out_ref, comm_ref, send_sems, recv_sems):
        my_x = lax.axis_index("x")
        my_y = lax.axis_index("y")
        mz = lax.axis_index("z")
        left = lax.rem(mz + NZ - 1, NZ)
        right = lax.rem(mz + 1, NZ)

        barrier_sem = pltpu.get_barrier_semaphore()
        for nbr in (left, right):
            pl.semaphore_signal(
                barrier_sem, inc=1,
                device_id=(my_x, my_y, nbr),
                device_id_type=pl.DeviceIdType.MESH,
            )
        pl.semaphore_wait(barrier_sem, 2)

        out_ref[...] = x_ref[...]

        for h in range(NZ - 1):
            src = x_ref if h == 0 else comm_ref.at[h - 1]
            rdma = pltpu.make_async_remote_copy(
                src_ref=src,
                dst_ref=comm_ref.at[h],
                send_sem=send_sems.at[h],
                recv_sem=recv_sems.at[h],
                device_id=(my_x, my_y, right),
                device_id_type=pl.DeviceIdType.MESH,
            )
            rdma.start()
            rdma.wait()
            out_ref[...] += comm_ref[h]

    return pl.pallas_call(
        body,
        out_shape=jax.ShapeDtypeStruct((t, d), jnp.float32),
        in_specs=[pl.BlockSpec(memory_space=pltpu.VMEM)],
        out_specs=pl.BlockSpec(memory_space=pltpu.VMEM),
        scratch_shapes=[
            pltpu.VMEM((NZ - 1, t, d), jnp.float32),
            pltpu.SemaphoreType.DMA((NZ - 1,)),
            pltpu.SemaphoreType.DMA((NZ - 1,)),
        ],
        compiler_params=pltpu.CompilerParams(collective_id=0),
    )(partial)


# device time: 179592 ns/iter; 7.7217x vs baseline; 2.0654x over previous
import functools

import jax
import jax.numpy as jnp
from jax import lax
from jax.experimental import pallas as pl
from jax.experimental.pallas import tpu as pltpu

NZ = 4
NQ = 4
K = 16


def _gather_partial(safe, msk, E, tq, d):

    def body(idx_ref, msk_ref, e_ref, o_ref, sems):
        o_ref[...] = jnp.zeros_like(o_ref)

        def cp(tk):
            return pltpu.make_async_copy(
                e_ref.at[idx_ref[tk]], o_ref.at[tk], sems.at[lax.rem(tk, K)]
            )

        def loop(tk, c):
            @pl.when((tk >= K) & (msk_ref[tk - K] == 1))
            def _():
                cp(tk - K).wait()

            @pl.when(msk_ref[tk] == 1)
            def _():
                cp(tk).start()

            return c

        lax.fori_loop(0, tq, loop, 0)

        def ep(i, c):
            tk = tq - K + i

            @pl.when(msk_ref[tk] == 1)
            def _():
                cp(tk).wait()

            return c

        lax.fori_loop(0, K, ep, 0)

    return pl.pallas_call(
        body,
        out_shape=jax.ShapeDtypeStruct((tq, d), jnp.float32),
        in_specs=[
            pl.BlockSpec(memory_space=pltpu.SMEM),
            pl.BlockSpec(memory_space=pltpu.SMEM),
            pl.BlockSpec(memory_space=pl.ANY),
        ],
        out_specs=pl.BlockSpec(memory_space=pltpu.VMEM),
        scratch_shapes=[pltpu.SemaphoreType.DMA((K,))],
    )(safe, msk, E)


def _xy_coords(p):
    px = p // 2
    return px, px ^ (p % 2)


def _comm(partial, t, d):
    tq = t // NQ

    def body(x_ref, out_ref, acc_ref, comm_a, sa, ra, comm_b, sb, rb):
        mx = lax.axis_index("x")
        my = lax.axis_index("y")
        mz = lax.axis_index("z")
        r = 2 * mx + (mx ^ my)
        zr = lax.rem(mz + 1, NZ)
        zl = lax.rem(mz + NZ - 1, NZ)
        rxr, ryr = _xy_coords(lax.rem(r + 1, NQ))
        rxl, ryl = _xy_coords(lax.rem(r + NQ - 1, NQ))

        neighbors = ((mx, my, zl), (mx, my, zr), (rxl, ryl, mz), (rxr, ryr, mz))

        barrier_sem = pltpu.get_barrier_semaphore()
        for nbr in neighbors:
            pl.semaphore_signal(
                barrier_sem, inc=1,
                device_id=nbr, device_id_type=pl.DeviceIdType.MESH,
            )
        pl.semaphore_wait(barrier_sem, len(neighbors))

        acc_ref[...] = x_ref[...]

        for h in range(NZ - 1):
            src = x_ref if h == 0 else comm_a.at[h - 1]
            rdma = pltpu.make_async_remote_copy(
                src_ref=src,
                dst_ref=comm_a.at[h],
                send_sem=sa.at[h],
                recv_sem=ra.at[h],
                device_id=(mx, my, zr),
                device_id_type=pl.DeviceIdType.MESH,
            )
            rdma.start()
            rdma.wait()
            acc_ref[...] += comm_a[h]

        out_ref[pl.ds(r * tq, tq), :] = acc_ref[...]

        for h in range(NQ - 1):
            src = acc_ref if h == 0 else comm_b.at[h - 1]
            rdma = pltpu.make_async_remote_copy(
                src_ref=src,
                dst_ref=comm_b.at[h],
                send_sem=sb.at[h],
                recv_sem=rb.at[h],
                device_id=(rxr, ryr, mz),
                device_id_type=pl.DeviceIdType.MESH,
            )
            rdma.start()
            rdma.wait()
            origin = lax.rem(r + NQ - 1 - h, NQ)
            out_ref[pl.ds(origin * tq, tq), :] = comm_b[h]

        @functools.partial(
            pl.run_scoped, exit_sem=pltpu.SemaphoreType.REGULAR
        )
        def _(exit_sem):
            for nbr in neighbors:
                pl.semaphore_signal(
                    exit_sem, inc=1,
                    device_id=nbr, device_id_type=pl.DeviceIdType.MESH,
                )
            pl.semaphore_wait(exit_sem, len(neighbors))

    return pl.pallas_call(
        body,
        out_shape=jax.ShapeDtypeStruct((t, d), jnp.float32),
        in_specs=[pl.BlockSpec(memory_space=pltpu.VMEM)],
        out_specs=pl.BlockSpec(memory_space=pltpu.VMEM),
        scratch_shapes=[
            pltpu.VMEM((t // NQ, d), jnp.float32),
            pltpu.VMEM((NZ - 1, t // NQ, d), jnp.float32),
            pltpu.SemaphoreType.DMA((NZ - 1,)),
            pltpu.SemaphoreType.DMA((NZ - 1,)),
            pltpu.VMEM((NQ - 1, t // NQ, d), jnp.float32),
            pltpu.SemaphoreType.DMA((NQ - 1,)),
            pltpu.SemaphoreType.DMA((NQ - 1,)),
        ],
        compiler_params=pltpu.CompilerParams(collective_id=0),
    )(partial)


def kernel(ids, E):
    v_local, d = E.shape
    t = ids.shape[0]
    tq = t // NQ

    mx = lax.axis_index("x")
    my = lax.axis_index("y")
    mz = lax.axis_index("z")
    r = 2 * mx + (mx ^ my)

    ids_q = lax.dynamic_slice(ids, (r * tq,), (tq,))
    local = ids_q - mz * v_local
    in_range = (local >= 0) & (local < v_local)
    safe = jnp.where(in_range, local, 0).astype(jnp.int32)
    msk = in_range.astype(jnp.int32)

    partial = _gather_partial(safe, msk, E, tq, d)
    return _comm(partial, t, d)


# device time: 143768 ns/iter; 9.6458x vs baseline; 1.2492x over previous
import functools

import jax
import jax.numpy as jnp
from jax import lax
from jax.experimental import pallas as pl
from jax.experimental.pallas import tpu as pltpu

NZ = 4
NQ = 4
K = 16


def _gather_partial(safe, msk, E, tq, d):

    def body(idx_ref, msk_ref, e_ref, o_ref, sems):
        o_ref[...] = jnp.zeros_like(o_ref)

        def cp(tk):
            return pltpu.make_async_copy(
                e_ref.at[idx_ref[tk]], o_ref.at[tk], sems.at[lax.rem(tk, K)]
            )

        def loop(tk, c):
            @pl.when((tk >= K) & (msk_ref[tk - K] == 1))
            def _():
                cp(tk - K).wait()

            @pl.when(msk_ref[tk] == 1)
            def _():
                cp(tk).start()

            return c

        lax.fori_loop(0, tq, loop, 0)

        def ep(i, c):
            tk = tq - K + i

            @pl.when(msk_ref[tk] == 1)
            def _():
                cp(tk).wait()

            return c

        lax.fori_loop(0, K, ep, 0)

    return pl.pallas_call(
        body,
        out_shape=jax.ShapeDtypeStruct((tq, d), jnp.float32),
        in_specs=[
            pl.BlockSpec(memory_space=pltpu.SMEM),
            pl.BlockSpec(memory_space=pltpu.SMEM),
            pl.BlockSpec(memory_space=pl.ANY),
        ],
        out_specs=pl.BlockSpec(memory_space=pltpu.VMEM),
        scratch_shapes=[pltpu.SemaphoreType.DMA((K,))],
    )(safe, msk, E)


def _xy_coords(p):
    px = p // 2
    return px, px ^ (p % 2)


def _comm(partial, t, d):
    tq = t // NQ
    th = tq // 2

    def body(x_ref, out_ref, acc_ref, comm_a, sa, ra,
             b_l0, b_r0, b_l1, b_r1, sb, rb):
        mx = lax.axis_index("x")
        my = lax.axis_index("y")
        mz = lax.axis_index("z")
        r = 2 * mx + (mx ^ my)
        zr = lax.rem(mz + 1, NZ)
        zl = lax.rem(mz + NZ - 1, NZ)
        rxr, ryr = _xy_coords(lax.rem(r + 1, NQ))
        rxl, ryl = _xy_coords(lax.rem(r + NQ - 1, NQ))
        right = (rxr, ryr, mz)
        left = (rxl, ryl, mz)

        neighbors = ((mx, my, zl), (mx, my, zr), left, right)

        barrier_sem = pltpu.get_barrier_semaphore()
        for nbr in neighbors:
            pl.semaphore_signal(
                barrier_sem, inc=1,
                device_id=nbr, device_id_type=pl.DeviceIdType.MESH,
            )
        pl.semaphore_wait(barrier_sem, len(neighbors))

        acc_ref[...] = x_ref[...]

        a_rdmas = []
        for h in range(NZ - 1):
            src = x_ref if h == 0 else comm_a.at[h - 1]
            rdma = pltpu.make_async_remote_copy(
                src_ref=src,
                dst_ref=comm_a.at[h],
                send_sem=sa.at[h],
                recv_sem=ra.at[h],
                device_id=(mx, my, zr),
                device_id_type=pl.DeviceIdType.MESH,
            )
            rdma.start()
            a_rdmas.append(rdma)
            rdma.wait_recv()
            if h > 0:
                acc_ref[...] += comm_a[h - 1]
        acc_ref[...] += comm_a[NZ - 2]

        def bsend(src, dst, i, dev):
            return pltpu.make_async_remote_copy(
                src_ref=src, dst_ref=dst, send_sem=sb.at[i],
                recv_sem=rb.at[i], device_id=dev,
                device_id_type=pl.DeviceIdType.MESH,
            )

        r1r = bsend(acc_ref, b_l0, 0, right)
        r1l = bsend(acc_ref, b_r0, 1, left)
        r1r.start()
        r1l.start()
        out_ref[pl.ds(r * tq, tq), :] = acc_ref[...]
        r1r.wait_recv()
        r1l.wait_recv()

        r2r = bsend(b_l0.at[pl.ds(0, th)], b_l1, 2, right)
        r2l = bsend(b_r0.at[pl.ds(th, th)], b_r1, 3, left)
        r2r.start()
        r2l.start()
        o_l = lax.rem(r + NQ - 1, NQ)
        o_r = lax.rem(r + 1, NQ)
        o_d = lax.rem(r + 2, NQ)
        out_ref[pl.ds(o_l * tq, tq), :] = b_l0[...]
        out_ref[pl.ds(o_r * tq, tq), :] = b_r0[...]
        r2r.wait_recv()
        r2l.wait_recv()
        out_ref[pl.ds(o_d * tq, th), :] = b_l1[...]
        out_ref[pl.ds(o_d * tq + th, th), :] = b_r1[...]

        for rdma in a_rdmas + [r1r, r1l, r2r, r2l]:
            rdma.wait_send()

        @functools.partial(
            pl.run_scoped, exit_sem=pltpu.SemaphoreType.REGULAR
        )
        def _(exit_sem):
            for nbr in neighbors:
                pl.semaphore_signal(
                    exit_sem, inc=1,
                    device_id=nbr, device_id_type=pl.DeviceIdType.MESH,
                )
            pl.semaphore_wait(exit_sem, len(neighbors))

    return pl.pallas_call(
        body,
        out_shape=jax.ShapeDtypeStruct((t, d), jnp.float32),
        in_specs=[pl.BlockSpec(memory_space=pltpu.VMEM)],
        out_specs=pl.BlockSpec(memory_space=pltpu.VMEM),
        scratch_shapes=[
            pltpu.VMEM((tq, d), jnp.float32),
            pltpu.VMEM((NZ - 1, tq, d), jnp.float32),
            pltpu.SemaphoreType.DMA((NZ - 1,)),
            pltpu.SemaphoreType.DMA((NZ - 1,)),
            pltpu.VMEM((tq, d), jnp.float32),
            pltpu.VMEM((tq, d), jnp.float32),
            pltpu.VMEM((th, d), jnp.float32),
            pltpu.VMEM((th, d), jnp.float32),
            pltpu.SemaphoreType.DMA((4,)),
            pltpu.SemaphoreType.DMA((4,)),
        ],
        compiler_params=pltpu.CompilerParams(collective_id=0),
    )(partial)


def kernel(ids, E):
    v_local, d = E.shape
    t = ids.shape[0]
    tq = t // NQ

    mx = lax.axis_index("x")
    my = lax.axis_index("y")
    mz = lax.axis_index("z")
    r = 2 * mx + (mx ^ my)

    ids_q = lax.dynamic_slice(ids, (r * tq,), (tq,))
    local = ids_q - mz * v_local
    in_range = (local >= 0) & (local < v_local)
    safe = jnp.where(in_range, local, 0).astype(jnp.int32)
    msk = in_range.astype(jnp.int32)

    partial = _gather_partial(safe, msk, E, tq, d)
    return _comm(partial, t, d)


# device time: 127302 ns/iter; 10.8934x vs baseline; 1.1293x over previous
import functools

import jax
import jax.numpy as jnp
from jax import lax
from jax.experimental import pallas as pl
from jax.experimental.pallas import tpu as pltpu

NZ = 4
NQ = 4
K = 16
S = 2


def _gather_partial(safe, msk, E, tq, d):

    def body(idx_ref, msk_ref, e_ref, o_ref, sems):
        o_ref[...] = jnp.zeros_like(o_ref)

        def cp(tk):
            return pltpu.make_async_copy(
                e_ref.at[idx_ref[tk]], o_ref.at[tk], sems.at[lax.rem(tk, K)]
            )

        def loop(tk, c):
            @pl.when((tk >= K) & (msk_ref[tk - K] == 1))
            def _():
                cp(tk - K).wait()

            @pl.when(msk_ref[tk] == 1)
            def _():
                cp(tk).start()

            return c

        lax.fori_loop(0, tq, loop, 0)

        def ep(i, c):
            tk = tq - K + i

            @pl.when(msk_ref[tk] == 1)
            def _():
                cp(tk).wait()

            return c

        lax.fori_loop(0, K, ep, 0)

    return pl.pallas_call(
        body,
        out_shape=jax.ShapeDtypeStruct((tq, d), jnp.float32),
        in_specs=[
            pl.BlockSpec(memory_space=pltpu.SMEM),
            pl.BlockSpec(memory_space=pltpu.SMEM),
            pl.BlockSpec(memory_space=pl.ANY),
        ],
        out_specs=pl.BlockSpec(memory_space=pltpu.VMEM),
        scratch_shapes=[pltpu.SemaphoreType.DMA((K,))],
    )(safe, msk, E)


def _xy_coords(p):
    px = p // 2
    return px, px ^ (p % 2)


def _comm(partial, t, d):
    tq = t // NQ

    def body(x_ref, out_ref, acc_ref, comm_a, sa, ra,
             b_l0, b_r0, b_l1, b_r1, sb, rb):
        mx = lax.axis_index("x")
        my = lax.axis_index("y")
        mz = lax.axis_index("z")
        r = 2 * mx + (mx ^ my)
        zr = lax.rem(mz + 1, NZ)
        zl = lax.rem(mz + NZ - 1, NZ)
        rxr, ryr = _xy_coords(lax.rem(r + 1, NQ))
        rxl, ryl = _xy_coords(lax.rem(r + NQ - 1, NQ))
        right = (rxr, ryr, mz)
        left = (rxl, ryl, mz)

        neighbors = ((mx, my, zl), (mx, my, zr), left, right)

        barrier_sem = pltpu.get_barrier_semaphore()
        for nbr in neighbors:
            pl.semaphore_signal(
                barrier_sem, inc=1,
                device_id=nbr, device_id_type=pl.DeviceIdType.MESH,
            )
        pl.semaphore_wait(barrier_sem, len(neighbors))

        o_l = lax.rem(r + NQ - 1, NQ)
        o_r = lax.rem(r + 1, NQ)
        o_d = lax.rem(r + 2, NQ)

        rows = tq // S
        th2 = rows // 2

        def a_rdma(h, s):
            sl = pl.ds(s * rows, rows)
            src = x_ref.at[sl] if h == 0 else comm_a.at[h - 1, sl]
            return pltpu.make_async_remote_copy(
                src_ref=src,
                dst_ref=comm_a.at[h, sl],
                send_sem=sa.at[h, s],
                recv_sem=ra.at[h, s],
                device_id=(mx, my, zr),
                device_id_type=pl.DeviceIdType.MESH,
            )

        def b_rdma(i, s, src, dst, dev):
            return pltpu.make_async_remote_copy(
                src_ref=src, dst_ref=dst, send_sem=sb.at[i, s],
                recv_sem=rb.at[i, s], device_id=dev,
                device_id_type=pl.DeviceIdType.MESH,
            )

        acc_ref[...] = x_ref[...]

        A = {}
        for s in range(S):
            A[0, s] = a_rdma(0, s)
            A[0, s].start()
        for h in range(1, NZ - 1):
            for s in range(S):
                sl = pl.ds(s * rows, rows)
                A[h - 1, s].wait_recv()
                A[h, s] = a_rdma(h, s)
                A[h, s].start()
                acc_ref[sl, :] += comm_a[h - 1, sl, :]

        B = {}
        for s in range(S):
            sl = pl.ds(s * rows, rows)
            A[NZ - 2, s].wait_recv()
            acc_ref[sl, :] += comm_a[NZ - 2, sl, :]
            B[0, s] = b_rdma(0, s, acc_ref.at[sl], b_l0.at[sl], right)
            B[1, s] = b_rdma(1, s, acc_ref.at[sl], b_r0.at[sl], left)
            B[0, s].start()
            B[1, s].start()
            out_ref[pl.ds(r * tq + s * rows, rows), :] = acc_ref[sl, :]
        for s in range(S):
            B[0, s].wait_recv()
            B[1, s].wait_recv()
            B[2, s] = b_rdma(
                2, s, b_l0.at[pl.ds(s * rows, th2)],
                b_l1.at[pl.ds(s * th2, th2)], right)
            B[3, s] = b_rdma(
                3, s, b_r0.at[pl.ds(s * rows + th2, th2)],
                b_r1.at[pl.ds(s * th2, th2)], left)
            B[2, s].start()
            B[3, s].start()
            out_ref[pl.ds(o_l * tq + s * rows, rows), :] = \
                b_l0[pl.ds(s * rows, rows), :]
            out_ref[pl.ds(o_r * tq + s * rows, rows), :] = \
                b_r0[pl.ds(s * rows, rows), :]
        for s in range(S):
            B[2, s].wait_recv()
            B[3, s].wait_recv()
            out_ref[pl.ds(o_d * tq + s * rows, th2), :] = \
                b_l1[pl.ds(s * th2, th2), :]
            out_ref[pl.ds(o_d * tq + s * rows + th2, th2), :] = \
                b_r1[pl.ds(s * th2, th2), :]

        for rdma in list(A.values()) + list(B.values()):
            rdma.wait_send()

        @functools.partial(
            pl.run_scoped, exit_sem=pltpu.SemaphoreType.REGULAR
        )
        def _(exit_sem):
            for nbr in neighbors:
                pl.semaphore_signal(
                    exit_sem, inc=1,
                    device_id=nbr, device_id_type=pl.DeviceIdType.MESH,
                )
            pl.semaphore_wait(exit_sem, len(neighbors))

    return pl.pallas_call(
        body,
        out_shape=jax.ShapeDtypeStruct((t, d), jnp.float32),
        in_specs=[pl.BlockSpec(memory_space=pltpu.VMEM)],
        out_specs=pl.BlockSpec(memory_space=pltpu.VMEM),
        scratch_shapes=[
            pltpu.VMEM((tq, d), jnp.float32),
            pltpu.VMEM((NZ - 1, tq, d), jnp.float32),
            pltpu.SemaphoreType.DMA((NZ - 1, S)),
            pltpu.SemaphoreType.DMA((NZ - 1, S)),
            pltpu.VMEM((tq, d), jnp.float32),
            pltpu.VMEM((tq, d), jnp.float32),
            pltpu.VMEM((tq // 2, d), jnp.float32),
            pltpu.VMEM((tq // 2, d), jnp.float32),
            pltpu.SemaphoreType.DMA((4, S)),
            pltpu.SemaphoreType.DMA((4, S)),
        ],
        compiler_params=pltpu.CompilerParams(collective_id=0),
    )(partial)


def kernel(ids, E):
    v_local, d = E.shape
    t = ids.shape[0]
    tq = t // NQ

    mx = lax.axis_index("x")
    my = lax.axis_index("y")
    mz = lax.axis_index("z")
    r = 2 * mx + (mx ^ my)

    ids_q = lax.dynamic_slice(ids, (r * tq,), (tq,))
    local = ids_q - mz * v_local
    in_range = (local >= 0) & (local < v_local)
    safe = jnp.where(in_range, local, 0).astype(jnp.int32)
    msk = in_range.astype(jnp.int32)

    partial = _gather_partial(safe, msk, E, tq, d)
    return _comm(partial, t, d)


# device time: 115045 ns/iter; 12.0540x vs baseline; 1.1065x over previous
import functools

import jax
import jax.numpy as jnp
from jax import lax
from jax.experimental import pallas as pl
from jax.experimental.pallas import tpu as pltpu

NZ = 4
NQ = 4
K = 16
S = 2


def _xy_coords(p):
    px = p // 2
    return px, px ^ (p % 2)


def kernel(ids, E):
    v_local, d = E.shape
    t = ids.shape[0]
    tq = t // NQ
    rows = tq // S
    th2 = rows // 2

    def body(ids_ref, e_ref, out_ref, x_ref, gsem, acc_ref,
             comm_a, sa, ra, b_l0, b_r0, b_l1, b_r1, sb, rb):
        mx = lax.axis_index("x")
        my = lax.axis_index("y")
        mz = lax.axis_index("z")
        r = 2 * mx + (mx ^ my)
        zr = lax.rem(mz + 1, NZ)
        zl = lax.rem(mz + NZ - 1, NZ)
        rxr, ryr = _xy_coords(lax.rem(r + 1, NQ))
        rxl, ryl = _xy_coords(lax.rem(r + NQ - 1, NQ))
        right = (rxr, ryr, mz)
        left = (rxl, ryl, mz)
        o_l = lax.rem(r + NQ - 1, NQ)
        o_r = lax.rem(r + 1, NQ)
        o_d = lax.rem(r + 2, NQ)

        neighbors = ((mx, my, zl), (mx, my, zr), left, right)

        base = mz * v_local
        goff = r * tq

        def owned(tk):
            loc = ids_ref[goff + tk] - base
            return loc, (loc >= 0) & (loc < v_local)

        def gcp(tk):
            loc, _ = owned(tk)
            return pltpu.make_async_copy(
                e_ref.at[loc], x_ref.at[tk], gsem.at[lax.rem(tk, K)]
            )

        def gather(s):
            lo, hi = s * rows, (s + 1) * rows

            def lp(tk, c):
                tkm = lax.max(tk - K, 0)
                _, ow_prev = owned(tkm)

                @pl.when((tk >= lo + K) & ow_prev)
                def _():
                    gcp(tkm).wait()

                _, ow = owned(tk)

                @pl.when(ow)
                def _():
                    gcp(tk).start()

                return c

            lax.fori_loop(lo, hi, lp, 0)

            def ep(i, c):
                tk = hi - K + i
                _, ow = owned(tk)

                @pl.when(ow)
                def _():
                    gcp(tk).wait()

                return c

            lax.fori_loop(0, K, ep, 0)

        def a_rdma(h, s):
            sl = pl.ds(s * rows, rows)
            src = x_ref.at[sl] if h == 0 else comm_a.at[h - 1, sl]
            return pltpu.make_async_remote_copy(
                src_ref=src,
                dst_ref=comm_a.at[h, sl],
                send_sem=sa.at[h, s],
                recv_sem=ra.at[h, s],
                device_id=(mx, my, zr),
                device_id_type=pl.DeviceIdType.MESH,
            )

        def b_rdma(i, s, src, dst, dev):
            return pltpu.make_async_remote_copy(
                src_ref=src, dst_ref=dst, send_sem=sb.at[i, s],
                recv_sem=rb.at[i, s], device_id=dev,
                device_id_type=pl.DeviceIdType.MESH,
            )

        x_ref[...] = jnp.zeros_like(x_ref)
        barrier_sem = pltpu.get_barrier_semaphore()
        for nbr in neighbors:
            pl.semaphore_signal(
                barrier_sem, inc=1,
                device_id=nbr, device_id_type=pl.DeviceIdType.MESH,
            )
        gather(0)
        pl.semaphore_wait(barrier_sem, len(neighbors))

        A = {}
        A[0, 0] = a_rdma(0, 0)
        A[0, 0].start()
        for s in range(1, S):
            gather(s)
            A[0, s] = a_rdma(0, s)
            A[0, s].start()
        acc_ref[...] = x_ref[...]
        for h in range(1, NZ - 1):
            for s in range(S):
                sl = pl.ds(s * rows, rows)
                A[h - 1, s].wait_recv()
                A[h, s] = a_rdma(h, s)
                A[h, s].start()
                acc_ref[sl, :] += comm_a[h - 1, sl, :]

        B = {}
        for s in range(S):
            sl = pl.ds(s * rows, rows)
            A[NZ - 2, s].wait_recv()
            acc_ref[sl, :] += comm_a[NZ - 2, sl, :]
            B[0, s] = b_rdma(0, s, acc_ref.at[sl], b_l0.at[sl], right)
            B[1, s] = b_rdma(1, s, acc_ref.at[sl], b_r0.at[sl], left)
            B[0, s].start()
            B[1, s].start()
            out_ref[pl.ds(r * tq + s * rows, rows), :] = acc_ref[sl, :]
        for s in range(S):
            B[0, s].wait_recv()
            B[1, s].wait_recv()
            B[2, s] = b_rdma(
                2, s, b_l0.at[pl.ds(s * rows, th2)],
                b_l1.at[pl.ds(s * th2, th2)], right)
            B[3, s] = b_rdma(
                3, s, b_r0.at[pl.ds(s * rows + th2, th2)],
                b_r1.at[pl.ds(s * th2, th2)], left)
            B[2, s].start()
            B[3, s].start()
            out_ref[pl.ds(o_l * tq + s * rows, rows), :] = \
                b_l0[pl.ds(s * rows, rows), :]
            out_ref[pl.ds(o_r * tq + s * rows, rows), :] = \
                b_r0[pl.ds(s * rows, rows), :]
        for s in range(S):
            B[2, s].wait_recv()
            B[3, s].wait_recv()
            out_ref[pl.ds(o_d * tq + s * rows, th2), :] = \
                b_l1[pl.ds(s * th2, th2), :]
            out_ref[pl.ds(o_d * tq + s * rows + th2, th2), :] = \
                b_r1[pl.ds(s * th2, th2), :]

        for rdma in list(A.values()) + list(B.values()):
            rdma.wait_send()

        @functools.partial(
            pl.run_scoped, exit_sem=pltpu.SemaphoreType.REGULAR
        )
        def _(exit_sem):
            for nbr in neighbors:
                pl.semaphore_signal(
                    exit_sem, inc=1,
                    device_id=nbr, device_id_type=pl.DeviceIdType.MESH,
                )
            pl.semaphore_wait(exit_sem, len(neighbors))

    return pl.pallas_call(
        body,
        out_shape=jax.ShapeDtypeStruct((t, d), jnp.float32),
        in_specs=[
            pl.BlockSpec(memory_space=pltpu.SMEM),
            pl.BlockSpec(memory_space=pl.ANY),
        ],
        out_specs=pl.BlockSpec(memory_space=pltpu.VMEM),
        scratch_shapes=[
            pltpu.VMEM((tq, d), jnp.float32),
            pltpu.SemaphoreType.DMA((K,)),
            pltpu.VMEM((tq, d), jnp.float32),
            pltpu.VMEM((NZ - 1, tq, d), jnp.float32),
            pltpu.SemaphoreType.DMA((NZ - 1, S)),
            pltpu.SemaphoreType.DMA((NZ - 1, S)),
            pltpu.VMEM((tq, d), jnp.float32),
            pltpu.VMEM((tq, d), jnp.float32),
            pltpu.VMEM((tq // 2, d), jnp.float32),
            pltpu.VMEM((tq // 2, d), jnp.float32),
            pltpu.SemaphoreType.DMA((4, S)),
            pltpu.SemaphoreType.DMA((4, S)),
        ],
        compiler_params=pltpu.CompilerParams(collective_id=0),
    )(ids, E)


# device time: 105030 ns/iter; 13.2034x vs baseline; 1.0954x over previous
import functools

import jax
import jax.numpy as jnp
from jax import lax
from jax.experimental import pallas as pl
from jax.experimental.pallas import tpu as pltpu

NZ = 4
NQ = 4
K = 16
S = 4


def _xy_coords(p):
    px = p // 2
    return px, px ^ (p % 2)


def kernel(ids, E):
    v_local, d = E.shape
    t = ids.shape[0]
    tq = t // NQ
    rows = tq // S
    th2 = rows // 2

    def body(ids_ref, e_ref, out_ref, x_ref, gsem, acc_ref,
             comm_a, sa, ra, b_l0, b_r0, b_l1, b_r1, sb, rb):
        mx = lax.axis_index("x")
        my = lax.axis_index("y")
        mz = lax.axis_index("z")
        r = 2 * mx + (mx ^ my)
        zr = lax.rem(mz + 1, NZ)
        zl = lax.rem(mz + NZ - 1, NZ)
        rxr, ryr = _xy_coords(lax.rem(r + 1, NQ))
        rxl, ryl = _xy_coords(lax.rem(r + NQ - 1, NQ))
        right = (rxr, ryr, mz)
        left = (rxl, ryl, mz)
        o_l = lax.rem(r + NQ - 1, NQ)
        o_r = lax.rem(r + 1, NQ)
        o_d = lax.rem(r + 2, NQ)

        neighbors = ((mx, my, zl), (mx, my, zr), left, right)

        base = mz * v_local
        goff = r * tq

        def owned(tk):
            loc = ids_ref[goff + tk] - base
            return loc, (loc >= 0) & (loc < v_local)

        def gcp(tk):
            loc, _ = owned(tk)
            return pltpu.make_async_copy(
                e_ref.at[loc], x_ref.at[tk], gsem.at[lax.rem(tk, K)]
            )

        def gather(s):
            lo, hi = s * rows, (s + 1) * rows

            def lp(tk, c):
                tkm = lax.max(tk - K, 0)
                _, ow_prev = owned(tkm)

                @pl.when((tk >= lo + K) & ow_prev)
                def _():
                    gcp(tkm).wait()

                _, ow = owned(tk)

                @pl.when(ow)
                def _():
                    gcp(tk).start()

                return c

            lax.fori_loop(lo, hi, lp, 0)

            def ep(i, c):
                tk = hi - K + i
                _, ow = owned(tk)

                @pl.when(ow)
                def _():
                    gcp(tk).wait()

                return c

            lax.fori_loop(0, K, ep, 0)

        def a_rdma(h, s):
            sl = pl.ds(s * rows, rows)
            src = x_ref.at[sl] if h == 0 else comm_a.at[h - 1, sl]
            return pltpu.make_async_remote_copy(
                src_ref=src,
                dst_ref=comm_a.at[h, sl],
                send_sem=sa.at[h, s],
                recv_sem=ra.at[h, s],
                device_id=(mx, my, zr),
                device_id_type=pl.DeviceIdType.MESH,
            )

        def b_rdma(i, s, src, dst, dev):
            return pltpu.make_async_remote_copy(
                src_ref=src, dst_ref=dst, send_sem=sb.at[i, s],
                recv_sem=rb.at[i, s], device_id=dev,
                device_id_type=pl.DeviceIdType.MESH,
            )

        x_ref[...] = jnp.zeros_like(x_ref)
        barrier_sem = pltpu.get_barrier_semaphore()
        for nbr in neighbors:
            pl.semaphore_signal(
                barrier_sem, inc=1,
                device_id=nbr, device_id_type=pl.DeviceIdType.MESH,
            )
        gather(0)
        pl.semaphore_wait(barrier_sem, len(neighbors))

        A = {}
        A[0, 0] = a_rdma(0, 0)
        A[0, 0].start()
        for s in range(1, S):
            gather(s)
            A[0, s] = a_rdma(0, s)
            A[0, s].start()
        acc_ref[...] = x_ref[...]
        for h in range(1, NZ - 1):
            for s in range(S):
                sl = pl.ds(s * rows, rows)
                A[h - 1, s].wait_recv()
                A[h, s] = a_rdma(h, s)
                A[h, s].start()
                acc_ref[sl, :] += comm_a[h - 1, sl, :]

        B = {}
        for s in range(S):
            sl = pl.ds(s * rows, rows)
            A[NZ - 2, s].wait_recv()
            acc_ref[sl, :] += comm_a[NZ - 2, sl, :]
            B[0, s] = b_rdma(0, s, acc_ref.at[sl], b_l0.at[sl], right)
            B[1, s] = b_rdma(1, s, acc_ref.at[sl], b_r0.at[sl], left)
            B[0, s].start()
            B[1, s].start()
            out_ref[pl.ds(r * tq + s * rows, rows), :] = acc_ref[sl, :]
        for s in range(S):
            B[0, s].wait_recv()
            B[1, s].wait_recv()
            B[2, s] = b_rdma(
                2, s, b_l0.at[pl.ds(s * rows, th2)],
                b_l1.at[pl.ds(s * th2, th2)], right)
            B[3, s] = b_rdma(
                3, s, b_r0.at[pl.ds(s * rows + th2, th2)],
                b_r1.at[pl.ds(s * th2, th2)], left)
            B[2, s].start()
            B[3, s].start()
            out_ref[pl.ds(o_l * tq + s * rows, rows), :] = \
                b_l0[pl.ds(s * rows, rows), :]
            out_ref[pl.ds(o_r * tq + s * rows, rows), :] = \
                b_r0[pl.ds(s * rows, rows), :]
        for s in range(S):
            B[2, s].wait_recv()
            B[3, s].wait_recv()
            out_ref[pl.ds(o_d * tq + s * rows, th2), :] = \
                b_l1[pl.ds(s * th2, th2), :]
            out_ref[pl.ds(o_d * tq + s * rows + th2, th2), :] = \
                b_r1[pl.ds(s * th2, th2), :]

        for rdma in list(A.values()) + list(B.values()):
            rdma.wait_send()

        @functools.partial(
            pl.run_scoped, exit_sem=pltpu.SemaphoreType.REGULAR
        )
        def _(exit_sem):
            for nbr in neighbors:
                pl.semaphore_signal(
                    exit_sem, inc=1,
                    device_id=nbr, device_id_type=pl.DeviceIdType.MESH,
                )
            pl.semaphore_wait(exit_sem, len(neighbors))

    return pl.pallas_call(
        body,
        out_shape=jax.ShapeDtypeStruct((t, d), jnp.float32),
        in_specs=[
            pl.BlockSpec(memory_space=pltpu.SMEM),
            pl.BlockSpec(memory_space=pl.ANY),
        ],
        out_specs=pl.BlockSpec(memory_space=pltpu.VMEM),
        scratch_shapes=[
            pltpu.VMEM((tq, d), jnp.float32),
            pltpu.SemaphoreType.DMA((K,)),
            pltpu.VMEM((tq, d), jnp.float32),
            pltpu.VMEM((NZ - 1, tq, d), jnp.float32),
            pltpu.SemaphoreType.DMA((NZ - 1, S)),
            pltpu.SemaphoreType.DMA((NZ - 1, S)),
            pltpu.VMEM((tq, d), jnp.float32),
            pltpu.VMEM((tq, d), jnp.float32),
            pltpu.VMEM((tq // 2, d), jnp.float32),
            pltpu.VMEM((tq // 2, d), jnp.float32),
            pltpu.SemaphoreType.DMA((4, S)),
            pltpu.SemaphoreType.DMA((4, S)),
        ],
        compiler_params=pltpu.CompilerParams(collective_id=0),
    )(ids, E)


# device time: 100373 ns/iter; 13.8160x vs baseline; 1.0464x over previous
import functools

import jax
import jax.numpy as jnp
from jax import lax
from jax.experimental import pallas as pl
from jax.experimental.pallas import tpu as pltpu

NZ = 4
NQ = 4
K = 16
S = 8


def _xy_coords(p):
    px = p // 2
    return px, px ^ (p % 2)


def kernel(ids, E):
    v_local, d = E.shape
    t = ids.shape[0]
    tq = t // NQ
    rows = tq // S
    th2 = rows // 2

    def body(ids_ref, e_ref, out_ref, x_ref, gsem, acc_ref,
             comm_a, sa, ra, b_l0, b_r0, b_l1, b_r1, sb, rb):
        mx = lax.axis_index("x")
        my = lax.axis_index("y")
        mz = lax.axis_index("z")
        r = 2 * mx + (mx ^ my)
        zr = lax.rem(mz + 1, NZ)
        zl = lax.rem(mz + NZ - 1, NZ)
        rxr, ryr = _xy_coords(lax.rem(r + 1, NQ))
        rxl, ryl = _xy_coords(lax.rem(r + NQ - 1, NQ))
        right = (rxr, ryr, mz)
        left = (rxl, ryl, mz)
        o_l = lax.rem(r + NQ - 1, NQ)
        o_r = lax.rem(r + 1, NQ)
        o_d = lax.rem(r + 2, NQ)

        neighbors = ((mx, my, zl), (mx, my, zr), left, right)

        base = mz * v_local
        goff = r * tq

        def owned(tk):
            loc = ids_ref[goff + tk] - base
            return loc, (loc >= 0) & (loc < v_local)

        def gcp(tk):
            loc, _ = owned(tk)
            return pltpu.make_async_copy(
                e_ref.at[loc], x_ref.at[tk], gsem.at[lax.rem(tk, K)]
            )

        def gather(s):
            lo, hi = s * rows, (s + 1) * rows

            def lp(tk, c):
                tkm = lax.max(tk - K, 0)
                _, ow_prev = owned(tkm)

                @pl.when((tk >= lo + K) & ow_prev)
                def _():
                    gcp(tkm).wait()

                _, ow = owned(tk)

                @pl.when(ow)
                def _():
                    gcp(tk).start()

                return c

            lax.fori_loop(lo, hi, lp, 0)

            def ep(i, c):
                tk = hi - K + i
                _, ow = owned(tk)

                @pl.when(ow)
                def _():
                    gcp(tk).wait()

                return c

            lax.fori_loop(0, K, ep, 0)

        def a_rdma(h, s):
            sl = pl.ds(s * rows, rows)
            src = x_ref.at[sl] if h == 0 else comm_a.at[h - 1, sl]
            return pltpu.make_async_remote_copy(
                src_ref=src,
                dst_ref=comm_a.at[h, sl],
                send_sem=sa.at[h, s],
                recv_sem=ra.at[h, s],
                device_id=(mx, my, zr),
                device_id_type=pl.DeviceIdType.MESH,
            )

        def b_rdma(i, s, src, dst, dev):
            return pltpu.make_async_remote_copy(
                src_ref=src, dst_ref=dst, send_sem=sb.at[i, s],
                recv_sem=rb.at[i, s], device_id=dev,
                device_id_type=pl.DeviceIdType.MESH,
            )

        x_ref[...] = jnp.zeros_like(x_ref)
        barrier_sem = pltpu.get_barrier_semaphore()
        for nbr in neighbors:
            pl.semaphore_signal(
                barrier_sem, inc=1,
                device_id=nbr, device_id_type=pl.DeviceIdType.MESH,
            )
        gather(0)
        pl.semaphore_wait(barrier_sem, len(neighbors))

        A = {}
        A[0, 0] = a_rdma(0, 0)
        A[0, 0].start()
        for s in range(1, S):
            gather(s)
            A[0, s] = a_rdma(0, s)
            A[0, s].start()
        acc_ref[...] = x_ref[...]
        for h in range(1, NZ - 1):
            for s in range(S):
                sl = pl.ds(s * rows, rows)
                A[h - 1, s].wait_recv()
                A[h, s] = a_rdma(h, s)
                A[h, s].start()
                acc_ref[sl, :] += comm_a[h - 1, sl, :]

        B = {}
        for s in range(S):
            sl = pl.ds(s * rows, rows)
            A[NZ - 2, s].wait_recv()
            acc_ref[sl, :] += comm_a[NZ - 2, sl, :]
            B[0, s] = b_rdma(0, s, acc_ref.at[sl], b_l0.at[sl], right)
            B[1, s] = b_rdma(1, s, acc_ref.at[sl], b_r0.at[sl], left)
            B[0, s].start()
            B[1, s].start()
            out_ref[pl.ds(r * tq + s * rows, rows), :] = acc_ref[sl, :]
        for s in range(S):
            B[0, s].wait_recv()
            B[1, s].wait_recv()
            B[2, s] = b_rdma(
                2, s, b_l0.at[pl.ds(s * rows, th2)],
                b_l1.at[pl.ds(s * th2, th2)], right)
            B[3, s] = b_rdma(
                3, s, b_r0.at[pl.ds(s * rows + th2, th2)],
                b_r1.at[pl.ds(s * th2, th2)], left)
            B[2, s].start()
            B[3, s].start()
            out_ref[pl.ds(o_l * tq + s * rows, rows), :] = \
                b_l0[pl.ds(s * rows, rows), :]
            out_ref[pl.ds(o_r * tq + s * rows, rows), :] = \
                b_r0[pl.ds(s * rows, rows), :]
        for s in range(S):
            B[2, s].wait_recv()
            B[3, s].wait_recv()
            out_ref[pl.ds(o_d * tq + s * rows, th2), :] = \
                b_l1[pl.ds(s * th2, th2), :]
            out_ref[pl.ds(o_d * tq + s * rows + th2, th2), :] = \
                b_r1[pl.ds(s * th2, th2), :]

        for rdma in list(A.values()) + list(B.values()):
            rdma.wait_send()

        @functools.partial(
            pl.run_scoped, exit_sem=pltpu.SemaphoreType.REGULAR
        )
        def _(exit_sem):
            for nbr in neighbors:
                pl.semaphore_signal(
                    exit_sem, inc=1,
                    device_id=nbr, device_id_type=pl.DeviceIdType.MESH,
                )
            pl.semaphore_wait(exit_sem, len(neighbors))

    return pl.pallas_call(
        body,
        out_shape=jax.ShapeDtypeStruct((t, d), jnp.float32),
        in_specs=[
            pl.BlockSpec(memory_space=pltpu.SMEM),
            pl.BlockSpec(memory_space=pl.ANY),
        ],
        out_specs=pl.BlockSpec(memory_space=pltpu.VMEM),
        scratch_shapes=[
            pltpu.VMEM((tq, d), jnp.float32),
            pltpu.SemaphoreType.DMA((K,)),
            pltpu.VMEM((tq, d), jnp.float32),
            pltpu.VMEM((NZ - 1, tq, d), jnp.float32),
            pltpu.SemaphoreType.DMA((NZ - 1, S)),
            pltpu.SemaphoreType.DMA((NZ - 1, S)),
            pltpu.VMEM((tq, d), jnp.float32),
            pltpu.VMEM((tq, d), jnp.float32),
            pltpu.VMEM((tq // 2, d), jnp.float32),
            pltpu.VMEM((tq // 2, d), jnp.float32),
            pltpu.SemaphoreType.DMA((4, S)),
            pltpu.SemaphoreType.DMA((4, S)),
        ],
        compiler_params=pltpu.CompilerParams(collective_id=0),
    )(ids, E)


# device time: 92109 ns/iter; 15.0556x vs baseline; 1.0897x over previous
import functools

import jax
import jax.numpy as jnp
from jax import lax
from jax.experimental import pallas as pl
from jax.experimental.pallas import tpu as pltpu

NZ = 4
NQ = 4
K = 16


def _xy_coords(p):
    px = p // 2
    return px, px ^ (p % 2)


def kernel(ids, E):
    v_local, d = E.shape
    t = ids.shape[0]
    tq = t // NQ
    sr = tq // NZ
    th2 = sr // 2

    def body(ids_ref, e_ref, out_ref, x_ref, gsem, acc_ref,
             rs_buf, srs, rrs, sag, rag,
             b_l0, b_r0, b_l1, b_r1, sb, rb):
        mx = lax.axis_index("x")
        my = lax.axis_index("y")
        mz = lax.axis_index("z")
        r = 2 * mx + (mx ^ my)
        rxr, ryr = _xy_coords(lax.rem(r + 1, NQ))
        rxl, ryl = _xy_coords(lax.rem(r + NQ - 1, NQ))
        right = (rxr, ryr, mz)
        left = (rxl, ryl, mz)
        o_l = lax.rem(r + NQ - 1, NQ)
        o_r = lax.rem(r + 1, NQ)
        o_d = lax.rem(r + 2, NQ)

        zpeers = [lax.rem(mz + k, NZ) for k in (1, 2, 3)]
        neighbors = tuple((mx, my, g) for g in zpeers) + (left, right)

        def seg_sl(g):
            return pl.ds(g * sr, sr)

        base = mz * v_local
        goff = r * tq

        def owned(tk):
            loc = ids_ref[goff + tk] - base
            return loc, (loc >= 0) & (loc < v_local)

        def gcp(tk):
            loc, _ = owned(tk)
            return pltpu.make_async_copy(
                e_ref.at[loc], x_ref.at[tk], gsem.at[lax.rem(tk, K)]
            )

        def gather_seg(g):
            lo = g * sr

            def lp(i, c):
                tk = lo + i
                tkm = lo + lax.max(i - K, 0)
                _, ow_prev = owned(tkm)

                @pl.when((i >= K) & ow_prev)
                def _():
                    gcp(tkm).wait()

                _, ow = owned(tk)

                @pl.when(ow)
                def _():
                    gcp(tk).start()

                return c

            lax.fori_loop(0, sr, lp, 0)

            def ep(i, c):
                tk = lo + sr - K + i
                _, ow = owned(tk)

                @pl.when(ow)
                def _():
                    gcp(tk).wait()

                return c

            lax.fori_loop(0, K, ep, 0)

        x_ref[...] = jnp.zeros_like(x_ref)
        barrier_sem = pltpu.get_barrier_semaphore()
        for nbr in neighbors:
            pl.semaphore_signal(
                barrier_sem, inc=1,
                device_id=nbr, device_id_type=pl.DeviceIdType.MESH,
            )
        gather_seg(zpeers[0])
        pl.semaphore_wait(barrier_sem, len(neighbors))

        sends = []

        for k, g in enumerate(zpeers):
            rdma = pltpu.make_async_remote_copy(
                src_ref=x_ref.at[seg_sl(g)],
                dst_ref=rs_buf.at[mz],
                send_sem=srs.at[k],
                recv_sem=rrs.at[mz],
                device_id=(mx, my, g),
                device_id_type=pl.DeviceIdType.MESH,
            )
            rdma.start()
            sends.append(rdma)
            gather_seg(zpeers[k + 1] if k < 2 else mz)

        for g in zpeers:
            pltpu.make_async_remote_copy(
                src_ref=rs_buf.at[g], dst_ref=rs_buf.at[g],
                send_sem=srs.at[0], recv_sem=rrs.at[g],
                device_id=(mx, my, g),
                device_id_type=pl.DeviceIdType.MESH,
            ).wait_recv()
        own = seg_sl(mz)
        acc_ref[own, :] = (
            x_ref[own, :] + rs_buf[zpeers[0], :, :]
            + rs_buf[zpeers[1], :, :] + rs_buf[zpeers[2], :, :]
        )

        for k, g in enumerate(zpeers):
            rdma = pltpu.make_async_remote_copy(
                src_ref=acc_ref.at[own],
                dst_ref=acc_ref.at[own],
                send_sem=sag.at[k],
                recv_sem=rag.at[mz],
                device_id=(mx, my, g),
                device_id_type=pl.DeviceIdType.MESH,
            )
            rdma.start()
            sends.append(rdma)

        def b_rdma(i, g, src, dst, dev):
            return pltpu.make_async_remote_copy(
                src_ref=src, dst_ref=dst, send_sem=sb.at[i, g],
                recv_sem=rb.at[i, g], device_id=dev,
                device_id_type=pl.DeviceIdType.MESH,
            )

        B = {}
        bsegs = [mz] + zpeers
        for n, g in enumerate(bsegs):
            sl = seg_sl(g)
            if n > 0:
                pltpu.make_async_remote_copy(
                    src_ref=acc_ref.at[sl], dst_ref=acc_ref.at[sl],
                    send_sem=sag.at[0], recv_sem=rag.at[g],
                    device_id=(mx, my, g),
                    device_id_type=pl.DeviceIdType.MESH,
                ).wait_recv()
            B[0, n] = b_rdma(0, g, acc_ref.at[sl], b_l0.at[sl], right)
            B[1, n] = b_rdma(1, g, acc_ref.at[sl], b_r0.at[sl], left)
            B[0, n].start()
            B[1, n].start()
            out_ref[pl.ds(r * tq + g * sr, sr), :] = acc_ref[sl, :]
        for n, g in enumerate(bsegs):
            B[0, n].wait_recv()
            B[1, n].wait_recv()
            B[2, n] = b_rdma(
                2, g, b_l0.at[pl.ds(g * sr, th2)],
                b_l1.at[pl.ds(g * th2, th2)], right)
            B[3, n] = b_rdma(
                3, g, b_r0.at[pl.ds(g * sr + th2, th2)],
                b_r1.at[pl.ds(g * th2, th2)], left)
            B[2, n].start()
            B[3, n].start()
            out_ref[pl.ds(o_l * tq + g * sr, sr), :] = \
                b_l0[pl.ds(g * sr, sr), :]
            out_ref[pl.ds(o_r * tq + g * sr, sr), :] = \
                b_r0[pl.ds(g * sr, sr), :]
        for n, g in enumerate(bsegs):
            B[2, n].wait_recv()
            B[3, n].wait_recv()
            out_ref[pl.ds(o_d * tq + g * sr, th2), :] = \
                b_l1[pl.ds(g * th2, th2), :]
            out_ref[pl.ds(o_d * tq + g * sr + th2, th2), :] = \
                b_r1[pl.ds(g * th2, th2), :]

        for rdma in sends + list(B.values()):
            rdma.wait_send()

        @functools.partial(
            pl.run_scoped, exit_sem=pltpu.SemaphoreType.REGULAR
        )
        def _(exit_sem):
            for nbr in neighbors:
                pl.semaphore_signal(
                    exit_sem, inc=1,
                    device_id=nbr, device_id_type=pl.DeviceIdType.MESH,
                )
            pl.semaphore_wait(exit_sem, len(neighbors))

    return pl.pallas_call(
        body,
        out_shape=jax.ShapeDtypeStruct((t, d), jnp.float32),
        in_specs=[
            pl.BlockSpec(memory_space=pltpu.SMEM),
            pl.BlockSpec(memory_space=pl.ANY),
        ],
        out_specs=pl.BlockSpec(memory_space=pltpu.VMEM),
        scratch_shapes=[
            pltpu.VMEM((tq, d), jnp.float32),
            pltpu.SemaphoreType.DMA((K,)),
            pltpu.VMEM((tq, d), jnp.float32),
            pltpu.VMEM((NZ, sr, d), jnp.float32),
            pltpu.SemaphoreType.DMA((NZ - 1,)),
            pltpu.SemaphoreType.DMA((NZ,)),
            pltpu.SemaphoreType.DMA((NZ - 1,)),
            pltpu.SemaphoreType.DMA((NZ,)),
            pltpu.VMEM((tq, d), jnp.float32),
            pltpu.VMEM((tq, d), jnp.float32),
            pltpu.VMEM((tq // 2, d), jnp.float32),
            pltpu.VMEM((tq // 2, d), jnp.float32),
            pltpu.SemaphoreType.DMA((4, NZ)),
            pltpu.SemaphoreType.DMA((4, NZ)),
        ],
        compiler_params=pltpu.CompilerParams(collective_id=0),
    )(ids, E)
